# baseline (speedup 1.0000x reference)
import jax
import jax.numpy as jnp
from jax import lax
from jax.experimental import pallas as pl
from jax.experimental.pallas import tpu as pltpu
from jax.experimental.pallas import tpu_sc as plsc

EMBED_DIM = 64
HIST = 50
NUM_CORES = 2
NUM_SUBCORES = 16
NUM_WORKERS = NUM_CORES * NUM_SUBCORES
BPW = 512
CB = 128
QPW = BPW // CB
NCH = QPW * HIST
LANES = 16
SKEW = EMBED_DIM + 1


def _gather_body(x_hbm, table_hbm, out_hbm,
                 idx_v, idx_t, g0, g1, s0, s1, t0, t1, gs0, gs1, ws0, ws1):
    wid = lax.axis_index("s") * NUM_CORES + lax.axis_index("c")
    base_b = wid * BPW

    gbuf = (g0, g1)
    sbuf = (s0, s1)
    tbuf = (t0, t1)
    gsems = (gs0, gs1)
    wsems = (ws0, ws1)

    pltpu.sync_copy(x_hbm.at[pl.ds(base_b * HIST, BPW * HIST)], idx_v)

    iota = lax.iota(jnp.int32, LANES)
    iota_h = iota * HIST

    @plsc.parallel_loop(0, HIST, unroll=2)
    def idx_h(h):
        for g in range(BPW // LANES):
            src = iota_h + (g * LANES * HIST + h)
            vals = plsc.load_gather(idx_v, [src])
            idx_t[pl.ds(h * BPW + g * LANES, LANES)] = vals

    def gather_start(c, b):
        pltpu.async_copy(
            table_hbm.at[idx_t.at[pl.ds(c * CB, CB)]], gbuf[b], gsems[b])

    def gather_wait(b):
        pltpu.make_async_copy(
            table_hbm.at[idx_t.at[pl.ds(0, CB)]], gbuf[b], gsems[b]).wait()

    def write_start(c, b):
        h = c // QPW
        tc0 = (base_b + (c % QPW) * CB) // 128
        pltpu.async_copy(
            tbuf[b], out_hbm.at[h, :, pl.ds(tc0, CB // 128)], wsems[b])

    def write_wait(b):
        pltpu.make_async_copy(
            tbuf[b], out_hbm.at[0, :, pl.ds(0, CB // 128)], wsems[b]).wait()

    def transpose_chunk(b):
        gsrc = gbuf[b]
        ssrc = sbuf[b]
        tdst = tbuf[b]

        @plsc.parallel_loop(0, CB, unroll=16)
        def skew_body(bb):
            for k in range(EMBED_DIM // LANES):
                ssrc[bb, pl.ds(k * LANES, LANES)] = (
                    gsrc[bb, pl.ds(k * LANES, LANES)])

        @plsc.parallel_loop(0, 8, unroll=8)
        def tbody(r):
            for cg in range(CB // LANES):
                rows = iota + cg * LANES
                for tr in range(EMBED_DIM // 8):
                    cols = jnp.zeros((LANES,), jnp.int32) + (8 * tr + r)
                    vals = plsc.load_gather(ssrc, [rows, cols])
                    tdst[tr, 0, r, pl.ds(cg * LANES, LANES)] = vals

    gather_start(0, 0)
    gather_start(1, 1)

    for b in range(2):
        gather_wait(b)
        transpose_chunk(b)
        write_start(b, b)
        gather_start(b + 2, b)

    def pair(g, carry):
        c = 2 * g
        for b in range(2):
            gather_wait(b)
            write_wait(b)
            transpose_chunk(b)
            write_start(c + b, b)
            gather_start(c + b + 2, b)
        return carry

    lax.fori_loop(1, NCH // 2 - 1, pair, 0)

    for b in range(2):
        gather_wait(b)
        write_wait(b)
        transpose_chunk(b)
        write_start(NCH - 2 + b, b)
        write_wait(b)


@jax.jit
def kernel(x, embedding):
    batch, hist = x.shape
    total = batch * hist
    xf = x.reshape(total)
    gather = pl.kernel(
        _gather_body,
        mesh=plsc.VectorSubcoreMesh(core_axis_name="c", subcore_axis_name="s"),
        out_type=jax.ShapeDtypeStruct(
            (hist, EMBED_DIM // 8, batch // 128, 8, 128), jnp.float32),
        compiler_params=pltpu.CompilerParams(
            use_tc_tiling_on_sc=False, needs_layout_passes=False),
        scratch_types=[
            pltpu.VMEM((BPW * HIST,), jnp.int32),
            pltpu.VMEM((NCH * CB,), jnp.int32),
            pltpu.VMEM((CB, EMBED_DIM), jnp.float32),
            pltpu.VMEM((CB, EMBED_DIM), jnp.float32),
            pltpu.VMEM((CB, SKEW), jnp.float32),
            pltpu.VMEM((CB, SKEW), jnp.float32),
            pltpu.VMEM((EMBED_DIM // 8, CB // 128, 8, 128), jnp.float32),
            pltpu.VMEM((EMBED_DIM // 8, CB // 128, 8, 128), jnp.float32),
            pltpu.SemaphoreType.DMA,
            pltpu.SemaphoreType.DMA,
            pltpu.SemaphoreType.DMA,
            pltpu.SemaphoreType.DMA,
        ],
    )
    out_tiled = gather(xf, embedding)
    return out_tiled.transpose(2, 4, 0, 1, 3).reshape(batch, hist, EMBED_DIM)


# speedup vs baseline: 1.2953x; 1.2953x over previous
import jax
import jax.numpy as jnp
from jax import lax
from jax.experimental import pallas as pl
from jax.experimental.pallas import tpu as pltpu
from jax.experimental.pallas import tpu_sc as plsc

EMBED_DIM = 64
HIST = 50
NUM_CORES = 2
NUM_SUBCORES = 16
NUM_WORKERS = NUM_CORES * NUM_SUBCORES
BPW = 512
CB = 128
QPW = BPW // CB
NCH = QPW * HIST
LANES = 16
SKEW = EMBED_DIM + 1


def _gather_body(x_hbm, table_hbm, out_hbm,
                 idx_v, idx_t, g0, g1, s0, s1, t0, t1, gs0, gs1, ws0, ws1):
    wid = lax.axis_index("s") * NUM_CORES + lax.axis_index("c")
    base_b = wid * BPW

    gbuf = (g0, g1)
    sbuf = (s0, s1)
    tbuf = (t0, t1)
    gsems = (gs0, gs1)
    wsems = (ws0, ws1)

    pltpu.sync_copy(x_hbm.at[pl.ds(base_b * HIST, BPW * HIST)], idx_v)

    iota = lax.iota(jnp.int32, LANES)
    iota_h = iota * HIST

    @plsc.parallel_loop(0, HIST, unroll=2)
    def idx_h(h):
        for g in range(BPW // LANES):
            src = iota_h + (g * LANES * HIST + h)
            vals = plsc.load_gather(idx_v, [src])
            idx_t[pl.ds(h * BPW + g * LANES, LANES)] = vals

    def gather_start(c, b):
        pltpu.async_copy(
            table_hbm.at[idx_t.at[pl.ds(c * CB, CB)]], gbuf[b], gsems[b])

    def gather_wait(b):
        pltpu.make_async_copy(
            table_hbm.at[idx_t.at[pl.ds(0, CB)]], gbuf[b], gsems[b]).wait()

    def write_start(c, b):
        h = c // QPW
        tc0 = (base_b + (c % QPW) * CB) // 128
        pltpu.async_copy(
            tbuf[b], out_hbm.at[h, :, pl.ds(tc0, CB // 128)], wsems[b])

    def write_wait(b):
        pltpu.make_async_copy(
            tbuf[b], out_hbm.at[0, :, pl.ds(0, CB // 128)], wsems[b]).wait()

    def transpose_chunk(b):
        gsrc = gbuf[b]
        ssrc = sbuf[b]
        tdst = tbuf[b]

        @plsc.parallel_loop(0, CB, unroll=8)
        def skew_body(bb):
            for k in range(EMBED_DIM // LANES):
                ssrc[bb, pl.ds(k * LANES, LANES)] = (
                    gsrc[bb, pl.ds(k * LANES, LANES)])

        @plsc.parallel_loop(0, 8, unroll=4)
        def tbody(r):
            for cg in range(CB // LANES):
                rows = iota + cg * LANES
                for tr in range(EMBED_DIM // 8):
                    cols = jnp.zeros((LANES,), jnp.int32) + (8 * tr + r)
                    vals = plsc.load_gather(ssrc, [rows, cols])
                    tdst[tr, 0, r, pl.ds(cg * LANES, LANES)] = vals

    gather_start(0, 0)
    gather_start(1, 1)

    for b in range(2):
        gather_wait(b)
        transpose_chunk(b)
        write_start(b, b)
        gather_start(b + 2, b)

    def pair(g, carry):
        c = 2 * g
        for b in range(2):
            gather_wait(b)
            write_wait(b)
            transpose_chunk(b)
            write_start(c + b, b)
            gather_start(c + b + 2, b)
        return carry

    lax.fori_loop(1, NCH // 2 - 1, pair, 0)

    for b in range(2):
        gather_wait(b)
        write_wait(b)
        transpose_chunk(b)
        write_start(NCH - 2 + b, b)
        write_wait(b)


@jax.jit
def kernel(x, embedding):
    batch, hist = x.shape
    total = batch * hist
    xf = x.reshape(total)
    gather = pl.kernel(
        _gather_body,
        mesh=plsc.VectorSubcoreMesh(core_axis_name="c", subcore_axis_name="s"),
        out_type=jax.ShapeDtypeStruct(
            (hist, EMBED_DIM // 8, batch // 128, 8, 128), jnp.float32),
        compiler_params=pltpu.CompilerParams(
            use_tc_tiling_on_sc=False, needs_layout_passes=False),
        scratch_types=[
            pltpu.VMEM((BPW * HIST,), jnp.int32),
            pltpu.VMEM((NCH * CB,), jnp.int32),
            pltpu.VMEM((CB, EMBED_DIM), jnp.float32),
            pltpu.VMEM((CB, EMBED_DIM), jnp.float32),
            pltpu.VMEM((CB, SKEW), jnp.float32),
            pltpu.VMEM((CB, SKEW), jnp.float32),
            pltpu.VMEM((EMBED_DIM // 8, CB // 128, 8, 128), jnp.float32),
            pltpu.VMEM((EMBED_DIM // 8, CB // 128, 8, 128), jnp.float32),
            pltpu.SemaphoreType.DMA,
            pltpu.SemaphoreType.DMA,
            pltpu.SemaphoreType.DMA,
            pltpu.SemaphoreType.DMA,
        ],
    )
    out_tiled = gather(xf, embedding)
    return out_tiled.transpose(2, 4, 0, 1, 3).reshape(batch, hist, EMBED_DIM)
